# Initial kernel scaffold; baseline (speedup 1.0000x reference)
import jax
import jax.numpy as jnp
from jax.experimental import pallas as pl


def kernel(atomic_offset_energy, atom_type, batch_ids, cell, potential_bias, potential_std, potential_total):
    raise NotImplementedError("write your pallas kernel here")


# speedup vs baseline: 117.3164x; 117.3164x over previous
import functools
import jax
import jax.numpy as jnp
from jax import lax
from jax.experimental import pallas as pl
from jax.experimental.pallas import tpu as pltpu
from jax.experimental.pallas import tpu_sc as plsc


def _build(N, B, E):
    info = plsc.get_sparse_core_info()
    NC, NS, L = info.num_cores, info.num_subcores, info.num_lanes
    NW = NC * NS
    assert N % NW == 0
    CHUNK = N // NW
    S = 10000
    assert CHUNK % S == 0 and S % L == 0
    NSUB = CHUNK // S
    VPS = S // L
    SW = B // NS

    mesh = plsc.VectorSubcoreMesh(core_axis_name="c", subcore_axis_name="s")

    @functools.partial(
        pl.kernel,
        out_type=jax.ShapeDtypeStruct((NC, B), jnp.float32),
        mesh=mesh,
        compiler_params=pltpu.CompilerParams(needs_layout_passes=False),
        scratch_types=dict(
            bias_v=pltpu.VMEM((E,), jnp.float32),
            std_v=pltpu.VMEM((L,), jnp.float32),
            offs_v=pltpu.VMEM((S,), jnp.float32),
            type_v=pltpu.VMEM((S,), jnp.int32),
            bids_v=pltpu.VMEM((S,), jnp.int32),
            acc_v=pltpu.VMEM((B,), jnp.float32),
            tmp_v=pltpu.VMEM((SW,), jnp.float32),
            strip_v=pltpu.VMEM((SW,), jnp.float32),
            shared=pltpu.VMEM_SHARED((NS, B), jnp.float32),
        ),
    )
    def run(offs_hbm, type_hbm, bids_hbm, bias_hbm, std_hbm, out_hbm,
            bias_v, std_v, offs_v, type_v, bids_v, acc_v, tmp_v, strip_v,
            shared):
        cid = lax.axis_index("c")
        sid = lax.axis_index("s")
        wid = sid * NC + cid
        base = wid * CHUNK

        pltpu.sync_copy(bias_hbm, bias_v)
        pltpu.sync_copy(std_hbm, std_v.at[pl.ds(0, 1)])

        zero = jnp.zeros((L,), jnp.float32)
        for i in range(B // L):
            acc_v[pl.ds(i * L, L)] = zero

        std = std_v[...][0]

        def body(i, carry):
            sl = pl.ds(i * L, L)
            t = type_v[sl]
            o = offs_v[sl]
            ids = bids_v[sl]
            b = plsc.load_gather(bias_v, [t])
            e = o * std + b
            plsc.addupdate_scatter(acc_v, [ids], e)
            return carry

        for sub in range(NSUB):
            off = base + sub * S
            pltpu.sync_copy(offs_hbm.at[pl.ds(off, S)], offs_v)
            pltpu.sync_copy(type_hbm.at[pl.ds(off, S)], type_v)
            pltpu.sync_copy(bids_hbm.at[pl.ds(off, S)], bids_v)
            lax.fori_loop(0, VPS, body, 0)

        pltpu.sync_copy(acc_v, shared.at[sid])
        plsc.subcore_barrier()

        for k in range(SW // L):
            strip_v[pl.ds(k * L, L)] = zero
        for r in range(NS):
            pltpu.sync_copy(shared.at[r, pl.ds(sid * SW, SW)], tmp_v)
            for k in range(SW // L):
                sl = pl.ds(k * L, L)
                strip_v[sl] = strip_v[sl] + tmp_v[sl]
        pltpu.sync_copy(strip_v, out_hbm.at[cid, pl.ds(sid * SW, SW)])

    return run


def kernel(atomic_offset_energy, atom_type, batch_ids, cell, potential_bias,
           potential_std, potential_total):
    N = atomic_offset_energy.shape[0]
    B = cell.shape[0]
    E = potential_bias.shape[0]
    run = _build(N, B, E)
    partials = run(atomic_offset_energy, atom_type, batch_ids, potential_bias,
                   potential_std)
    return partials[0] + partials[1] + potential_total


# speedup vs baseline: 154.7835x; 1.3194x over previous
import functools
import jax
import jax.numpy as jnp
from jax import lax
from jax.experimental import pallas as pl
from jax.experimental.pallas import tpu as pltpu
from jax.experimental.pallas import tpu_sc as plsc


def _build(N, B, E):
    info = plsc.get_sparse_core_info()
    NC, NS, L = info.num_cores, info.num_subcores, info.num_lanes
    NW = NC * NS
    assert N % NW == 0
    CHUNK = N // NW
    S = 10000
    assert CHUNK % S == 0 and S % L == 0
    NSUB = CHUNK // S
    VPS = S // L
    SW = B // NS

    mesh = plsc.VectorSubcoreMesh(core_axis_name="c", subcore_axis_name="s")

    @functools.partial(
        pl.kernel,
        out_type=jax.ShapeDtypeStruct((NC, B), jnp.float32),
        mesh=mesh,
        compiler_params=pltpu.CompilerParams(needs_layout_passes=False),
        scratch_types=dict(
            bias_v=pltpu.VMEM((E,), jnp.float32),
            std_v=pltpu.VMEM((L,), jnp.float32),
            offs_v0=pltpu.VMEM((S,), jnp.float32),
            offs_v1=pltpu.VMEM((S,), jnp.float32),
            type_v0=pltpu.VMEM((S,), jnp.int32),
            type_v1=pltpu.VMEM((S,), jnp.int32),
            bids_v0=pltpu.VMEM((S,), jnp.int32),
            bids_v1=pltpu.VMEM((S,), jnp.int32),
            acc_v=pltpu.VMEM((B,), jnp.float32),
            tmp_v=pltpu.VMEM((SW,), jnp.float32),
            strip_v=pltpu.VMEM((SW,), jnp.float32),
            shared=pltpu.VMEM_SHARED((NS, B), jnp.float32),
            sem=pltpu.SemaphoreType.DMA,
        ),
    )
    def run(offs_hbm, type_hbm, bids_hbm, bias_hbm, std_hbm, out_hbm,
            bias_v, std_v, offs_v0, offs_v1, type_v0, type_v1, bids_v0,
            bids_v1, acc_v, tmp_v, strip_v, shared, sem):
        offs_b = (offs_v0, offs_v1)
        type_b = (type_v0, type_v1)
        bids_b = (bids_v0, bids_v1)
        cid = lax.axis_index("c")
        sid = lax.axis_index("s")
        wid = sid * NC + cid
        base = wid * CHUNK

        pltpu.sync_copy(bias_hbm, bias_v)
        pltpu.sync_copy(std_hbm, std_v.at[pl.ds(0, 1)])

        zero = jnp.zeros((L,), jnp.float32)
        for i in range(B // L):
            acc_v[pl.ds(i * L, L)] = zero

        std = std_v[...][0]

        def start(sub):
            off = base + sub * S
            buf = sub % 2
            return [
                pltpu.async_copy(offs_hbm.at[pl.ds(off, S)], offs_b[buf], sem),
                pltpu.async_copy(type_hbm.at[pl.ds(off, S)], type_b[buf], sem),
                pltpu.async_copy(bids_hbm.at[pl.ds(off, S)], bids_b[buf], sem),
            ]

        descs = start(0)
        for sub in range(NSUB):
            buf = sub % 2
            for d in descs:
                d.wait()
            if sub + 1 < NSUB:
                descs = start(sub + 1)

            @plsc.parallel_loop(0, VPS, unroll=8)
            def body(i):
                sl = pl.ds(i * L, L)
                t = type_b[buf][sl]
                o = offs_b[buf][sl]
                ids = bids_b[buf][sl]
                b = plsc.load_gather(bias_v, [t])
                e = o * std + b
                plsc.addupdate_scatter(acc_v, [ids], e)

        pltpu.sync_copy(acc_v, shared.at[sid])
        plsc.subcore_barrier()

        for k in range(SW // L):
            strip_v[pl.ds(k * L, L)] = zero
        for r in range(NS):
            pltpu.sync_copy(shared.at[r, pl.ds(sid * SW, SW)], tmp_v)
            for k in range(SW // L):
                sl = pl.ds(k * L, L)
                strip_v[sl] = strip_v[sl] + tmp_v[sl]
        pltpu.sync_copy(strip_v, out_hbm.at[cid, pl.ds(sid * SW, SW)])

    return run


def kernel(atomic_offset_energy, atom_type, batch_ids, cell, potential_bias,
           potential_std, potential_total):
    N = atomic_offset_energy.shape[0]
    B = cell.shape[0]
    E = potential_bias.shape[0]
    run = _build(N, B, E)
    partials = run(atomic_offset_energy, atom_type, batch_ids, potential_bias,
                   potential_std)
    return partials[0] + partials[1] + potential_total


# speedup vs baseline: 342.6107x; 2.2135x over previous
import functools
import jax
import jax.numpy as jnp
from jax import lax
from jax.experimental import pallas as pl
from jax.experimental.pallas import tpu as pltpu
from jax.experimental.pallas import tpu_sc as plsc


def _build(N, B, E):
    info = plsc.get_sparse_core_info()
    NC, NS, L = info.num_cores, info.num_subcores, info.num_lanes
    NW = NC * NS
    assert N % NW == 0
    CHUNK = N // NW
    S = 10000
    assert CHUNK % S == 0 and S % L == 0
    NSUB = CHUNK // S
    VPS = S // L
    SW = B // NS

    mesh = plsc.VectorSubcoreMesh(core_axis_name="c", subcore_axis_name="s")

    @functools.partial(
        pl.kernel,
        out_type=jax.ShapeDtypeStruct((NC, B), jnp.float32),
        mesh=mesh,
        compiler_params=pltpu.CompilerParams(needs_layout_passes=False),
        scratch_types=dict(
            bias_v=pltpu.VMEM((E,), jnp.float32),
            std_v=pltpu.VMEM((L,), jnp.float32),
            offs_v0=pltpu.VMEM((S,), jnp.float32),
            offs_v1=pltpu.VMEM((S,), jnp.float32),
            type_v0=pltpu.VMEM((S,), jnp.int32),
            type_v1=pltpu.VMEM((S,), jnp.int32),
            bids_v0=pltpu.VMEM((S + 16,), jnp.int32),
            bids_v1=pltpu.VMEM((S + 16,), jnp.int32),
            acc_v=pltpu.VMEM((B,), jnp.float32),
            tmp_v=pltpu.VMEM((SW,), jnp.float32),
            strip_v=pltpu.VMEM((SW,), jnp.float32),
            shared=pltpu.VMEM_SHARED((NS, B), jnp.float32),
            sem=pltpu.SemaphoreType.DMA,
        ),
    )
    def run(offs_hbm, type_hbm, bids_hbm, bias_hbm, std_hbm, out_hbm,
            bias_v, std_v, offs_v0, offs_v1, type_v0, type_v1, bids_v0,
            bids_v1, acc_v, tmp_v, strip_v, shared, sem):
        offs_b = (offs_v0, offs_v1)
        type_b = (type_v0, type_v1)
        bids_b = (bids_v0, bids_v1)
        cid = lax.axis_index("c")
        sid = lax.axis_index("s")
        wid = sid * NC + cid
        base = wid * CHUNK

        pltpu.sync_copy(bias_hbm, bias_v)
        pltpu.sync_copy(std_hbm, std_v.at[pl.ds(0, 1)])

        zero = jnp.zeros((L,), jnp.float32)
        for i in range(B // L):
            acc_v[pl.ds(i * L, L)] = zero

        std = std_v[...][0]

        def start(sub):
            off = base + sub * S
            buf = sub % 2
            return [
                pltpu.async_copy(offs_hbm.at[pl.ds(off, S)], offs_b[buf], sem),
                pltpu.async_copy(type_hbm.at[pl.ds(off, S)], type_b[buf], sem),
                pltpu.async_copy(bids_hbm.at[pl.ds(off, S)],
                                 bids_b[buf].at[pl.ds(0, S)], sem),
            ]

        sent = jnp.full((L,), -1, jnp.int32)
        bids_v0[pl.ds(S, L)] = sent
        bids_v1[pl.ds(S, L)] = sent

        iota = lax.iota(jnp.int32, L)
        l15 = iota == (L - 1)
        notl15 = iota < (L - 1)

        descs = start(0)
        for sub in range(NSUB):
            buf = sub % 2
            for d in descs:
                d.wait()
            if sub + 1 < NSUB:
                descs = start(sub + 1)

            @plsc.parallel_loop(0, VPS, unroll=8)
            def body(i):
                sl = pl.ds(i * L, L)
                t = type_b[buf][sl]
                o = offs_b[buf][sl]
                ids = bids_b[buf][sl]
                ids_s = bids_b[buf][pl.ds(i * L + 1, L)]
                b = plsc.load_gather(bias_v, [t])
                e = o * std + b
                csum = plsc.cumsum(e)
                bm = ids != ids_s
                plsc.addupdate_scatter(acc_v, [ids], csum, mask=bm | l15)
                plsc.addupdate_scatter(acc_v, [ids_s], -csum,
                                       mask=bm & notl15)

        pltpu.sync_copy(acc_v, shared.at[sid])
        plsc.subcore_barrier()

        for k in range(SW // L):
            strip_v[pl.ds(k * L, L)] = zero
        for r in range(NS):
            pltpu.sync_copy(shared.at[r, pl.ds(sid * SW, SW)], tmp_v)
            for k in range(SW // L):
                sl = pl.ds(k * L, L)
                strip_v[sl] = strip_v[sl] + tmp_v[sl]
        pltpu.sync_copy(strip_v, out_hbm.at[cid, pl.ds(sid * SW, SW)])

    return run


def kernel(atomic_offset_energy, atom_type, batch_ids, cell, potential_bias,
           potential_std, potential_total):
    N = atomic_offset_energy.shape[0]
    B = cell.shape[0]
    E = potential_bias.shape[0]
    run = _build(N, B, E)
    partials = run(atomic_offset_energy, atom_type, batch_ids, potential_bias,
                   potential_std)
    return partials[0] + partials[1] + potential_total


# speedup vs baseline: 357.8553x; 1.0445x over previous
import functools
import jax
import jax.numpy as jnp
from jax import lax
from jax.experimental import pallas as pl
from jax.experimental.pallas import tpu as pltpu
from jax.experimental.pallas import tpu_sc as plsc


def _build(N, B, E):
    info = plsc.get_sparse_core_info()
    NC, NS, L = info.num_cores, info.num_subcores, info.num_lanes
    NW = NC * NS
    assert N % NW == 0
    CHUNK = N // NW
    S = 10000
    assert CHUNK % S == 0 and S % L == 0
    NSUB = CHUNK // S
    VPS = S // L
    SW = B // NS

    mesh = plsc.VectorSubcoreMesh(core_axis_name="c", subcore_axis_name="s")

    @functools.partial(
        pl.kernel,
        out_type=jax.ShapeDtypeStruct((NC, B), jnp.float32),
        mesh=mesh,
        compiler_params=pltpu.CompilerParams(needs_layout_passes=False),
        scratch_types=dict(
            bias_v=pltpu.VMEM((E,), jnp.float32),
            std_v=pltpu.VMEM((L,), jnp.float32),
            offs_v0=pltpu.VMEM((S,), jnp.float32),
            offs_v1=pltpu.VMEM((S,), jnp.float32),
            type_v0=pltpu.VMEM((S,), jnp.int32),
            type_v1=pltpu.VMEM((S,), jnp.int32),
            bids_v0=pltpu.VMEM((S + 16,), jnp.int32),
            bids_v1=pltpu.VMEM((S + 16,), jnp.int32),
            acc_v=pltpu.VMEM((B,), jnp.float32),
            tmp_v=pltpu.VMEM((SW,), jnp.float32),
            strip_v=pltpu.VMEM((SW,), jnp.float32),
            shared=pltpu.VMEM_SHARED((NS, B), jnp.float32),
            sem=pltpu.SemaphoreType.DMA,
        ),
    )
    def run(offs_hbm, type_hbm, bids_hbm, bias_hbm, std_hbm, out_hbm,
            bias_v, std_v, offs_v0, offs_v1, type_v0, type_v1, bids_v0,
            bids_v1, acc_v, tmp_v, strip_v, shared, sem):
        offs_b = (offs_v0, offs_v1)
        type_b = (type_v0, type_v1)
        bids_b = (bids_v0, bids_v1)
        cid = lax.axis_index("c")
        sid = lax.axis_index("s")
        wid = sid * NC + cid
        base = wid * CHUNK

        pltpu.sync_copy(bias_hbm, bias_v)
        pltpu.sync_copy(std_hbm, std_v.at[pl.ds(0, 1)])

        zero = jnp.zeros((L,), jnp.float32)
        for i in range(B // L):
            acc_v[pl.ds(i * L, L)] = zero

        std = std_v[...][0]

        def start(sub):
            off = base + sub * S
            buf = sub % 2
            return [
                pltpu.async_copy(offs_hbm.at[pl.ds(off, S)], offs_b[buf], sem),
                pltpu.async_copy(type_hbm.at[pl.ds(off, S)], type_b[buf], sem),
                pltpu.async_copy(bids_hbm.at[pl.ds(off, S)],
                                 bids_b[buf].at[pl.ds(0, S)], sem),
            ]

        sent = jnp.full((L,), -1, jnp.int32)
        bids_v0[pl.ds(S, L)] = sent
        bids_v1[pl.ds(S, L)] = sent

        iota = lax.iota(jnp.int32, L)
        l15 = iota == (L - 1)
        notl15 = iota < (L - 1)
        rot_keys = (iota + (L - 1)) % L

        descs = start(0)
        for sub in range(NSUB):
            buf = sub % 2
            for d in descs:
                d.wait()
            if sub + 1 < NSUB:
                descs = start(sub + 1)

            @plsc.parallel_loop(0, VPS, unroll=8)
            def body(i):
                sl = pl.ds(i * L, L)
                t = type_b[buf][sl]
                o = offs_b[buf][sl]
                ids = bids_b[buf][sl]
                _, ids_s = plsc.sort_key_val(rot_keys, ids)
                b = plsc.load_gather(bias_v, [t])
                e = o * std + b
                csum = plsc.cumsum(e)
                bm = ids != ids_s
                plsc.addupdate_scatter(acc_v, [ids], csum, mask=bm | l15)
                plsc.addupdate_scatter(acc_v, [ids_s], -csum,
                                       mask=bm & notl15)

        pltpu.sync_copy(acc_v, shared.at[sid])
        plsc.subcore_barrier()

        for k in range(SW // L):
            strip_v[pl.ds(k * L, L)] = zero
        for r in range(NS):
            pltpu.sync_copy(shared.at[r, pl.ds(sid * SW, SW)], tmp_v)
            for k in range(SW // L):
                sl = pl.ds(k * L, L)
                strip_v[sl] = strip_v[sl] + tmp_v[sl]
        pltpu.sync_copy(strip_v, out_hbm.at[cid, pl.ds(sid * SW, SW)])

    return run


def kernel(atomic_offset_energy, atom_type, batch_ids, cell, potential_bias,
           potential_std, potential_total):
    N = atomic_offset_energy.shape[0]
    B = cell.shape[0]
    E = potential_bias.shape[0]
    run = _build(N, B, E)
    partials = run(atomic_offset_energy, atom_type, batch_ids, potential_bias,
                   potential_std)
    return partials[0] + partials[1] + potential_total


# speedup vs baseline: 368.2881x; 1.0292x over previous
import functools
import jax
import jax.numpy as jnp
from jax import lax
from jax.experimental import pallas as pl
from jax.experimental.pallas import tpu as pltpu
from jax.experimental.pallas import tpu_sc as plsc


def _build(N, B, E):
    info = plsc.get_sparse_core_info()
    NC, NS, L = info.num_cores, info.num_subcores, info.num_lanes
    NW = NC * NS
    assert N % NW == 0
    CHUNK = N // NW
    S = 10000
    assert CHUNK % S == 0 and S % L == 0
    NSUB = CHUNK // S
    VPS = S // L
    SW = B // NS

    mesh = plsc.VectorSubcoreMesh(core_axis_name="c", subcore_axis_name="s")

    @functools.partial(
        pl.kernel,
        out_type=jax.ShapeDtypeStruct((NC, B), jnp.float32),
        mesh=mesh,
        compiler_params=pltpu.CompilerParams(needs_layout_passes=False),
        scratch_types=dict(
            bias_v=pltpu.VMEM((E,), jnp.float32),
            std_v=pltpu.VMEM((L,), jnp.float32),
            offs_v0=pltpu.VMEM((S,), jnp.float32),
            offs_v1=pltpu.VMEM((S,), jnp.float32),
            type_v0=pltpu.VMEM((S,), jnp.int32),
            type_v1=pltpu.VMEM((S,), jnp.int32),
            bids_v0=pltpu.VMEM((S,), jnp.int32),
            bids_v1=pltpu.VMEM((S,), jnp.int32),
            acc_p=pltpu.VMEM((B,), jnp.float32),
            acc_n=pltpu.VMEM((B,), jnp.float32),
            tmp_v=pltpu.VMEM((NS * SW,), jnp.float32),
            strip_v=pltpu.VMEM((SW,), jnp.float32),
            shared=pltpu.VMEM_SHARED((NS, B), jnp.float32),
            sem=pltpu.SemaphoreType.DMA,
        ),
    )
    def run(offs_hbm, type_hbm, bids_hbm, bias_hbm, std_hbm, out_hbm,
            bias_v, std_v, offs_v0, offs_v1, type_v0, type_v1, bids_v0,
            bids_v1, acc_p, acc_n, tmp_v, strip_v, shared, sem):
        offs_b = (offs_v0, offs_v1)
        type_b = (type_v0, type_v1)
        bids_b = (bids_v0, bids_v1)
        cid = lax.axis_index("c")
        sid = lax.axis_index("s")
        wid = sid * NC + cid
        base = wid * CHUNK

        pltpu.sync_copy(bias_hbm, bias_v)
        pltpu.sync_copy(std_hbm, std_v.at[pl.ds(0, 1)])

        zero = jnp.zeros((L,), jnp.float32)
        for i in range(B // L):
            acc_p[pl.ds(i * L, L)] = zero
            acc_n[pl.ds(i * L, L)] = zero

        std = std_v[...][0]

        def start(sub):
            off = base + sub * S
            buf = sub % 2
            return [
                pltpu.async_copy(offs_hbm.at[pl.ds(off, S)], offs_b[buf], sem),
                pltpu.async_copy(type_hbm.at[pl.ds(off, S)], type_b[buf], sem),
                pltpu.async_copy(bids_hbm.at[pl.ds(off, S)], bids_b[buf], sem),
            ]

        iota = lax.iota(jnp.int32, L)
        l15 = iota == (L - 1)
        notl15 = iota < (L - 1)
        rot_keys = (iota + (L - 1)) % L

        descs = start(0)
        for sub in range(NSUB):
            buf = sub % 2
            for d in descs:
                d.wait()
            if sub + 1 < NSUB:
                descs = start(sub + 1)

            @plsc.parallel_loop(0, VPS, unroll=8)
            def body(i):
                sl = pl.ds(i * L, L)
                t = type_b[buf][sl]
                o = offs_b[buf][sl]
                ids = bids_b[buf][sl]
                _, ids_s = plsc.sort_key_val(rot_keys, ids)
                b = plsc.load_gather(bias_v, [t])
                e = o * std + b
                csum = plsc.cumsum(e)
                bm = ids != ids_s
                plsc.addupdate_scatter(acc_p, [ids], csum, mask=bm | l15)
                plsc.addupdate_scatter(acc_n, [ids_s], csum,
                                       mask=bm & notl15)

        for i in range(B // L):
            sl = pl.ds(i * L, L)
            acc_p[sl] = acc_p[sl] - acc_n[sl]
        pltpu.sync_copy(acc_p, shared.at[sid])
        plsc.subcore_barrier()

        copies = [
            pltpu.async_copy(shared.at[r, pl.ds(sid * SW, SW)],
                             tmp_v.at[pl.ds(r * SW, SW)], sem)
            for r in range(NS)
        ]
        for d in copies:
            d.wait()
        for k in range(SW // L):
            strip_v[pl.ds(k * L, L)] = zero
        for r in range(NS):
            for k in range(SW // L):
                sl = pl.ds(k * L, L)
                strip_v[sl] = strip_v[sl] + tmp_v[pl.ds(r * SW + k * L, L)]
        pltpu.sync_copy(strip_v, out_hbm.at[cid, pl.ds(sid * SW, SW)])

    return run


def kernel(atomic_offset_energy, atom_type, batch_ids, cell, potential_bias,
           potential_std, potential_total):
    N = atomic_offset_energy.shape[0]
    B = cell.shape[0]
    E = potential_bias.shape[0]
    run = _build(N, B, E)
    partials = run(atomic_offset_energy, atom_type, batch_ids, potential_bias,
                   potential_std)
    return partials[0] + partials[1] + potential_total


# speedup vs baseline: 378.2809x; 1.0271x over previous
import functools
import jax
import jax.numpy as jnp
from jax import lax
from jax.experimental import pallas as pl
from jax.experimental.pallas import tpu as pltpu
from jax.experimental.pallas import tpu_sc as plsc


def _build(N, B, E):
    info = plsc.get_sparse_core_info()
    NC, NS, L = info.num_cores, info.num_subcores, info.num_lanes
    NW = NC * NS
    assert N % NW == 0
    CHUNK = N // NW
    S = 10000
    assert CHUNK % S == 0 and S % L == 0
    NSUB = CHUNK // S
    VPS = S // L
    SW = B // NS

    mesh = plsc.VectorSubcoreMesh(core_axis_name="c", subcore_axis_name="s")

    @functools.partial(
        pl.kernel,
        out_type=jax.ShapeDtypeStruct((NC, B), jnp.float32),
        mesh=mesh,
        compiler_params=pltpu.CompilerParams(needs_layout_passes=False),
        scratch_types=dict(
            bias_v=pltpu.VMEM((E,), jnp.float32),
            std_v=pltpu.VMEM((L,), jnp.float32),
            offs_v0=pltpu.VMEM((S,), jnp.float32),
            offs_v1=pltpu.VMEM((S,), jnp.float32),
            type_v0=pltpu.VMEM((S,), jnp.int32),
            type_v1=pltpu.VMEM((S,), jnp.int32),
            bids_v0=pltpu.VMEM((S,), jnp.int32),
            bids_v1=pltpu.VMEM((S,), jnp.int32),
            acc_p=pltpu.VMEM((B,), jnp.float32),
            acc_n=pltpu.VMEM((B,), jnp.float32),
            tmp_v=pltpu.VMEM((NS * SW,), jnp.float32),
            strip_v=pltpu.VMEM((SW,), jnp.float32),
            shared=pltpu.VMEM_SHARED((NS, B), jnp.float32),
            sem=pltpu.SemaphoreType.DMA,
        ),
    )
    def run(offs_hbm, type_hbm, bids_hbm, bias_hbm, std_hbm, out_hbm,
            bias_v, std_v, offs_v0, offs_v1, type_v0, type_v1, bids_v0,
            bids_v1, acc_p, acc_n, tmp_v, strip_v, shared, sem):
        offs_b = (offs_v0, offs_v1)
        type_b = (type_v0, type_v1)
        bids_b = (bids_v0, bids_v1)
        cid = lax.axis_index("c")
        sid = lax.axis_index("s")
        wid = sid * NC + cid
        base = wid * CHUNK

        def start(sub):
            off = base + sub * S
            buf = sub % 2
            return [
                pltpu.async_copy(offs_hbm.at[pl.ds(off, S)], offs_b[buf], sem),
                pltpu.async_copy(type_hbm.at[pl.ds(off, S)], type_b[buf], sem),
                pltpu.async_copy(bids_hbm.at[pl.ds(off, S)], bids_b[buf], sem),
            ]

        descs = start(0)

        pltpu.sync_copy(bias_hbm, bias_v)
        pltpu.sync_copy(std_hbm, std_v.at[pl.ds(0, 1)])

        zero = jnp.zeros((L,), jnp.float32)
        for i in range(B // L):
            acc_p[pl.ds(i * L, L)] = zero
            acc_n[pl.ds(i * L, L)] = zero

        std = std_v[...][0]

        iota = lax.iota(jnp.int32, L)
        l15 = iota == (L - 1)
        notl15 = iota < (L - 1)
        rot_keys = (iota + (L - 1)) % L

        for sub in range(NSUB):
            buf = sub % 2
            for d in descs:
                d.wait()
            if sub + 1 < NSUB:
                descs = start(sub + 1)

            @plsc.parallel_loop(0, VPS, unroll=8)
            def body(i):
                sl = pl.ds(i * L, L)
                t = type_b[buf][sl]
                o = offs_b[buf][sl]
                ids = bids_b[buf][sl]
                _, ids_s = plsc.sort_key_val(rot_keys, ids)
                b = plsc.load_gather(bias_v, [t])
                e = o * std + b
                csum = plsc.cumsum(e)
                bm = ids != ids_s
                plsc.addupdate_scatter(acc_p, [ids], csum, mask=bm | l15)
                plsc.addupdate_scatter(acc_n, [ids_s], csum,
                                       mask=bm & notl15)

        for i in range(B // L):
            sl = pl.ds(i * L, L)
            acc_p[sl] = acc_p[sl] - acc_n[sl]
        pltpu.sync_copy(acc_p, shared.at[sid])
        plsc.subcore_barrier()

        copies = [
            pltpu.async_copy(shared.at[r, pl.ds(sid * SW, SW)],
                             tmp_v.at[pl.ds(r * SW, SW)], sem)
            for r in range(NS)
        ]
        for d in copies:
            d.wait()
        for k in range(SW // L):
            strip_v[pl.ds(k * L, L)] = zero
        for r in range(NS):
            for k in range(SW // L):
                sl = pl.ds(k * L, L)
                strip_v[sl] = strip_v[sl] + tmp_v[pl.ds(r * SW + k * L, L)]
        pltpu.sync_copy(strip_v, out_hbm.at[cid, pl.ds(sid * SW, SW)])

    return run


def kernel(atomic_offset_energy, atom_type, batch_ids, cell, potential_bias,
           potential_std, potential_total):
    N = atomic_offset_energy.shape[0]
    B = cell.shape[0]
    E = potential_bias.shape[0]
    run = _build(N, B, E)
    partials = run(atomic_offset_energy, atom_type, batch_ids, potential_bias,
                   potential_std)
    return partials[0] + partials[1] + potential_total


# speedup vs baseline: 381.8633x; 1.0095x over previous
import functools
import jax
import jax.numpy as jnp
from jax import lax
from jax.experimental import pallas as pl
from jax.experimental.pallas import tpu as pltpu
from jax.experimental.pallas import tpu_sc as plsc


def _build(N, B, E):
    info = plsc.get_sparse_core_info()
    NC, NS, L = info.num_cores, info.num_subcores, info.num_lanes
    NW = NC * NS
    assert N % NW == 0
    CHUNK = N // NW
    S = 10000
    assert CHUNK % S == 0 and S % L == 0
    NSUB = CHUNK // S
    VPS = S // L
    SW = B // NS

    mesh = plsc.VectorSubcoreMesh(core_axis_name="c", subcore_axis_name="s")

    @functools.partial(
        pl.kernel,
        out_type=jax.ShapeDtypeStruct((NC, B), jnp.float32),
        mesh=mesh,
        compiler_params=pltpu.CompilerParams(needs_layout_passes=False),
        scratch_types=dict(
            bias_v=pltpu.VMEM((E,), jnp.float32),
            std_v=pltpu.VMEM((L,), jnp.float32),
            offs_v0=pltpu.VMEM((S,), jnp.float32),
            offs_v1=pltpu.VMEM((S,), jnp.float32),
            type_v0=pltpu.VMEM((S,), jnp.int32),
            type_v1=pltpu.VMEM((S,), jnp.int32),
            bids_v0=pltpu.VMEM((S,), jnp.int32),
            bids_v1=pltpu.VMEM((S,), jnp.int32),
            acc_p=pltpu.VMEM((B,), jnp.float32),
            acc_n=pltpu.VMEM((B,), jnp.float32),
            tmp_v=pltpu.VMEM((NS * SW,), jnp.float32),
            strip_v=pltpu.VMEM((SW,), jnp.float32),
            shared=pltpu.VMEM_SHARED((NS, B), jnp.float32),
            sem=pltpu.SemaphoreType.DMA,
        ),
    )
    def run(offs_hbm, type_hbm, bids_hbm, bias_hbm, std_hbm, out_hbm,
            bias_v, std_v, offs_v0, offs_v1, type_v0, type_v1, bids_v0,
            bids_v1, acc_p, acc_n, tmp_v, strip_v, shared, sem):
        offs_b = (offs_v0, offs_v1)
        type_b = (type_v0, type_v1)
        bids_b = (bids_v0, bids_v1)
        cid = lax.axis_index("c")
        sid = lax.axis_index("s")
        wid = sid * NC + cid
        base = wid * CHUNK

        def start(sub):
            off = base + sub * S
            buf = sub % 2
            return [
                pltpu.async_copy(offs_hbm.at[pl.ds(off, S)], offs_b[buf], sem),
                pltpu.async_copy(type_hbm.at[pl.ds(off, S)], type_b[buf], sem),
                pltpu.async_copy(bids_hbm.at[pl.ds(off, S)], bids_b[buf], sem),
            ]

        descs = start(0)

        pltpu.sync_copy(bias_hbm, bias_v)
        pltpu.sync_copy(std_hbm, std_v.at[pl.ds(0, 1)])

        zero = jnp.zeros((L,), jnp.float32)
        for i in range(B // L):
            acc_p[pl.ds(i * L, L)] = zero
            acc_n[pl.ds(i * L, L)] = zero

        std = std_v[...][0]

        iota = lax.iota(jnp.int32, L)
        l15 = iota == (L - 1)
        notl15 = iota < (L - 1)
        rot_keys = (iota + (L - 1)) % L

        for sub in range(NSUB):
            buf = sub % 2
            for d in descs:
                d.wait()
            if sub + 1 < NSUB:
                descs = start(sub + 1)

            @plsc.parallel_loop(0, VPS, unroll=5)
            def body(i):
                sl = pl.ds(i * L, L)
                t = type_b[buf][sl]
                o = offs_b[buf][sl]
                ids = bids_b[buf][sl]
                _, ids_s = plsc.sort_key_val(rot_keys, ids)
                b = plsc.load_gather(bias_v, [t])
                e = o * std + b
                csum = plsc.cumsum(e)
                bm = ids != ids_s
                plsc.addupdate_scatter(acc_p, [ids], csum, mask=bm | l15)
                plsc.addupdate_scatter(acc_n, [ids_s], csum,
                                       mask=bm & notl15)

        for i in range(B // L):
            sl = pl.ds(i * L, L)
            acc_p[sl] = acc_p[sl] - acc_n[sl]
        pltpu.sync_copy(acc_p, shared.at[sid])
        plsc.subcore_barrier()

        copies = [
            pltpu.async_copy(shared.at[r, pl.ds(sid * SW, SW)],
                             tmp_v.at[pl.ds(r * SW, SW)], sem)
            for r in range(NS)
        ]
        for d in copies:
            d.wait()
        for k in range(SW // L):
            strip_v[pl.ds(k * L, L)] = zero
        for r in range(NS):
            for k in range(SW // L):
                sl = pl.ds(k * L, L)
                strip_v[sl] = strip_v[sl] + tmp_v[pl.ds(r * SW + k * L, L)]
        pltpu.sync_copy(strip_v, out_hbm.at[cid, pl.ds(sid * SW, SW)])

    return run


def kernel(atomic_offset_energy, atom_type, batch_ids, cell, potential_bias,
           potential_std, potential_total):
    N = atomic_offset_energy.shape[0]
    B = cell.shape[0]
    E = potential_bias.shape[0]
    run = _build(N, B, E)
    partials = run(atomic_offset_energy, atom_type, batch_ids, potential_bias,
                   potential_std)
    return partials[0] + partials[1] + potential_total
